# Initial kernel scaffold; baseline (speedup 1.0000x reference)
import jax
import jax.numpy as jnp
from jax.experimental import pallas as pl


def kernel(x, W1, b1, W2, b2, W3, b3):
    raise NotImplementedError("write your pallas kernel here")


# speedup vs baseline: 16.1708x; 16.1708x over previous
import functools

import jax
import jax.numpy as jnp
import numpy as np
from jax import lax
from jax.experimental import pallas as pl
from jax.experimental.pallas import tpu as pltpu

SCALE = 2
K = 9
N = 8
H = 112
L = H * H
M = N * N
TL = 1568


def _sample_idx():
    ih = np.round(np.linspace(0.0, H - 1, N)).astype(np.int32)
    return (ih[:, None] * H + ih[None, :]).reshape(-1)


def _layer_body(x_ref, an_ref, w_ref, b_ref, o_ref, t_ref, *, out_c, relu):
    anchors = an_ref[0]

    @pl.when(pl.program_id(1) == 0)
    def _():
        for k in range(K):
            t_ref[k] = jnp.dot(anchors, w_ref[k],
                               preferred_element_type=jnp.float32)

    tile = x_ref[0]
    sim = lax.dot_general(tile, anchors, (((1,), (1,)), ((), ())),
                          preferred_element_type=jnp.float32)
    iota = lax.broadcasted_iota(jnp.int32, (TL, M), 1)
    acc = jnp.broadcast_to(b_ref[0], (TL, out_c)).astype(jnp.float32)
    for k in range(K):
        m = jnp.max(sim, axis=1, keepdims=True)
        cand = jnp.where(sim == m, iota, M)
        am = jnp.min(cand, axis=1, keepdims=True)
        sel = iota == am
        acc = acc + jnp.dot(sel.astype(jnp.float32), t_ref[k],
                            preferred_element_type=jnp.float32,
                            precision=lax.Precision.HIGHEST)
        sim = jnp.where(sel, -jnp.inf, sim)
    o_ref[0] = jnp.maximum(acc, 0.0) if relu else acc


def _layer(xT, anchors, W, b, relu, interpret=False):
    Bn, _, C = xT.shape
    O = W.shape[0]
    wk = jnp.transpose(W, (2, 1, 0))
    grid = (Bn, L // TL)
    return pl.pallas_call(
        functools.partial(_layer_body, out_c=O, relu=relu),
        grid=grid,
        in_specs=[
            pl.BlockSpec((1, TL, C), lambda b_, t: (b_, t, 0)),
            pl.BlockSpec((1, M, C), lambda b_, t: (b_, 0, 0)),
            pl.BlockSpec((K, C, O), lambda b_, t: (0, 0, 0)),
            pl.BlockSpec((1, O), lambda b_, t: (0, 0)),
        ],
        out_specs=pl.BlockSpec((1, TL, O), lambda b_, t: (b_, t, 0)),
        out_shape=jax.ShapeDtypeStruct((Bn, L, O), jnp.float32),
        scratch_shapes=[pltpu.VMEM((K, M, O), jnp.float32)],
        interpret=interpret,
    )(xT, anchors, wk, b.reshape(1, O))


def _run(x, W1, b1, W2, b2, W3, b3, interpret=False):
    B = x.shape[0]
    u = x.reshape(B, 3, H, SCALE, H, SCALE)
    u = u.transpose(0, 1, 3, 5, 2, 4).reshape(B, 12, H, H)
    xT = u.reshape(B, 12, L).transpose(0, 2, 1)
    sidx = _sample_idx()
    h = xT
    for W, b, relu in ((W1, b1, True), (W2, b2, True), (W3, b3, False)):
        anchors = h[:, sidx, :]
        h = _layer(h, anchors, W, b, relu, interpret=interpret)
    out = h.transpose(0, 2, 1).reshape(B, 12, H, H)
    out = out.reshape(B, 3, SCALE, SCALE, H, H)
    out = out.transpose(0, 1, 4, 2, 5, 3).reshape(B, 3, H * SCALE, H * SCALE)
    return out


def kernel(x, W1, b1, W2, b2, W3, b3):
    return _run(x, W1, b1, W2, b2, W3, b3)


# speedup vs baseline: 31.8893x; 1.9720x over previous
import functools

import jax
import jax.numpy as jnp
import numpy as np
from jax import lax
from jax.experimental import pallas as pl
from jax.experimental.pallas import tpu as pltpu

SCALE = 2
K = 9
N = 8
H = 112
L = H * H
M = N * N
TL = 1792


def _sample_idx():
    ih = np.round(np.linspace(0.0, H - 1, N)).astype(np.int32)
    return (ih[:, None] * H + ih[None, :]).reshape(-1)


def _layer_body(x_ref, an_ref, w_ref, b_ref, o_ref, t_ref, *, out_c, relu):
    anchors = an_ref[0]

    @pl.when(pl.program_id(1) == 0)
    def _():
        for k in range(K):
            t_ref[k] = jnp.dot(w_ref[k], anchors,
                               preferred_element_type=jnp.float32)

    tile = x_ref[0]
    sim = lax.dot_general(anchors, tile, (((0,), (0,)), ((), ())),
                          preferred_element_type=jnp.float32)
    iota = lax.broadcasted_iota(jnp.int32, (M, TL), 0).astype(jnp.float32)
    acc = jnp.broadcast_to(b_ref[...], (out_c, TL)).astype(jnp.float32)
    for k in range(K):
        m = jnp.max(sim, axis=0, keepdims=True)
        cand = jnp.where(sim == m, iota, float(M))
        am = jnp.min(cand, axis=0, keepdims=True)
        sel = iota == am
        acc = acc + jnp.dot(t_ref[k], sel.astype(jnp.float32),
                            preferred_element_type=jnp.float32,
                            precision=lax.Precision.HIGHEST)
        sim = jnp.where(sel, -jnp.inf, sim)
    o_ref[0] = jnp.maximum(acc, 0.0) if relu else acc


def _layer(xf, anchors, W, b, relu, interpret=False):
    Bn, C, _ = xf.shape
    O = W.shape[0]
    wk = jnp.transpose(W, (2, 0, 1))
    grid = (Bn, L // TL)
    return pl.pallas_call(
        functools.partial(_layer_body, out_c=O, relu=relu),
        grid=grid,
        in_specs=[
            pl.BlockSpec((1, C, TL), lambda b_, t: (b_, 0, t)),
            pl.BlockSpec((1, C, M), lambda b_, t: (b_, 0, 0)),
            pl.BlockSpec((K, O, C), lambda b_, t: (0, 0, 0)),
            pl.BlockSpec((O, 1), lambda b_, t: (0, 0)),
        ],
        out_specs=pl.BlockSpec((1, O, TL), lambda b_, t: (b_, 0, t)),
        out_shape=jax.ShapeDtypeStruct((Bn, O, L), jnp.float32),
        scratch_shapes=[pltpu.VMEM((K, O, M), jnp.float32)],
        interpret=interpret,
    )(xf, anchors, wk, b.reshape(O, 1))


def _run(x, W1, b1, W2, b2, W3, b3, interpret=False):
    B = x.shape[0]
    u = x.reshape(B, 3, H, SCALE, H, SCALE)
    u = u.transpose(0, 1, 3, 5, 2, 4).reshape(B, 12, L)
    sidx = _sample_idx()
    h = u
    for W, b, relu in ((W1, b1, True), (W2, b2, True), (W3, b3, False)):
        anchors = h[:, :, sidx]
        h = _layer(h, anchors, W, b, relu, interpret=interpret)
    out = h.reshape(B, 3, SCALE, SCALE, H, H)
    out = out.transpose(0, 1, 4, 2, 5, 3).reshape(B, 3, H * SCALE, H * SCALE)
    return out


def kernel(x, W1, b1, W2, b2, W3, b3):
    return _run(x, W1, b1, W2, b2, W3, b3)


# speedup vs baseline: 44.7660x; 1.4038x over previous
import functools

import jax
import jax.numpy as jnp
import numpy as np
from jax import lax
from jax.experimental import pallas as pl
from jax.experimental.pallas import tpu as pltpu

SCALE = 2
K = 9
N = 8
H = 112
L = H * H
M = N * N
TL = 1792


def _sample_idx():
    ih = np.round(np.linspace(0.0, H - 1, N)).astype(np.int32)
    return (ih[:, None] * H + ih[None, :]).reshape(-1)


def _layer_body(x_ref, an_ref, w_ref, b_ref, o_ref, t1_ref, t2_ref, t3_ref,
                *, out_c, relu):
    anchors = an_ref[0]

    @pl.when(pl.program_id(1) == 0)
    def _():
        for k in range(K):
            t = jnp.dot(w_ref[k], anchors,
                        preferred_element_type=jnp.float32)
            t1 = t.astype(jnp.bfloat16)
            r = t - t1.astype(jnp.float32)
            t2 = r.astype(jnp.bfloat16)
            t3 = (r - t2.astype(jnp.float32)).astype(jnp.bfloat16)
            t1_ref[k] = t1
            t2_ref[k] = t2
            t3_ref[k] = t3

    tile = x_ref[0]
    sim = lax.dot_general(anchors, tile, (((0,), (0,)), ((), ())),
                          preferred_element_type=jnp.float32)
    iota = lax.broadcasted_iota(jnp.int32, (M, TL), 0).astype(jnp.float32)
    acc = jnp.broadcast_to(b_ref[...], (out_c, TL)).astype(jnp.float32)
    for k in range(K):
        m = jnp.max(sim, axis=0, keepdims=True)
        cand = jnp.where(sim == m, iota, float(M))
        am = jnp.min(cand, axis=0, keepdims=True)
        sel = iota == am
        selb = sel.astype(jnp.bfloat16)
        for t_ref in (t1_ref, t2_ref, t3_ref):
            acc = acc + jnp.dot(t_ref[k], selb,
                                preferred_element_type=jnp.float32)
        sim = jnp.where(sel, -jnp.inf, sim)
    o_ref[0] = jnp.maximum(acc, 0.0) if relu else acc


def _layer(xf, anchors, W, b, relu, interpret=False):
    Bn, C, _ = xf.shape
    O = W.shape[0]
    wk = jnp.transpose(W, (2, 0, 1))
    grid = (Bn, L // TL)
    return pl.pallas_call(
        functools.partial(_layer_body, out_c=O, relu=relu),
        grid=grid,
        in_specs=[
            pl.BlockSpec((1, C, TL), lambda b_, t: (b_, 0, t)),
            pl.BlockSpec((1, C, M), lambda b_, t: (b_, 0, 0)),
            pl.BlockSpec((K, O, C), lambda b_, t: (0, 0, 0)),
            pl.BlockSpec((O, 1), lambda b_, t: (0, 0)),
        ],
        out_specs=pl.BlockSpec((1, O, TL), lambda b_, t: (b_, 0, t)),
        out_shape=jax.ShapeDtypeStruct((Bn, O, L), jnp.float32),
        scratch_shapes=[pltpu.VMEM((K, O, M), jnp.bfloat16),
                        pltpu.VMEM((K, O, M), jnp.bfloat16),
                        pltpu.VMEM((K, O, M), jnp.bfloat16)],
        interpret=interpret,
    )(xf, anchors, wk, b.reshape(O, 1))


def _run(x, W1, b1, W2, b2, W3, b3, interpret=False):
    B = x.shape[0]
    u = x.reshape(B, 3, H, SCALE, H, SCALE)
    u = u.transpose(0, 1, 3, 5, 2, 4).reshape(B, 12, L)
    sidx = _sample_idx()
    h = u
    for W, b, relu in ((W1, b1, True), (W2, b2, True), (W3, b3, False)):
        anchors = h[:, :, sidx]
        h = _layer(h, anchors, W, b, relu, interpret=interpret)
    out = h.reshape(B, 3, SCALE, SCALE, H, H)
    out = out.transpose(0, 1, 4, 2, 5, 3).reshape(B, 3, H * SCALE, H * SCALE)
    return out


def kernel(x, W1, b1, W2, b2, W3, b3):
    return _run(x, W1, b1, W2, b2, W3, b3)


# speedup vs baseline: 47.1100x; 1.0524x over previous
import functools

import jax
import jax.numpy as jnp
import numpy as np
from jax import lax
from jax.experimental import pallas as pl
from jax.experimental.pallas import tpu as pltpu

SCALE = 2
K = 9
N = 8
H = 112
L = H * H
M = N * N
TL = 6272


def _sample_idx():
    ih = np.round(np.linspace(0.0, H - 1, N)).astype(np.int32)
    return (ih[:, None] * H + ih[None, :]).reshape(-1)


def _layer_body(x_ref, an_ref, w_ref, b_ref, o_ref, t1_ref, t2_ref, t3_ref,
                *, out_c, relu):
    anchors = an_ref[0]

    @pl.when(pl.program_id(1) == 0)
    def _():
        for k in range(K):
            t = jnp.dot(w_ref[k], anchors,
                        preferred_element_type=jnp.float32)
            t1 = t.astype(jnp.bfloat16)
            r = t - t1.astype(jnp.float32)
            t2 = r.astype(jnp.bfloat16)
            t3 = (r - t2.astype(jnp.float32)).astype(jnp.bfloat16)
            t1_ref[k] = t1
            t2_ref[k] = t2
            t3_ref[k] = t3

    tile = x_ref[0]
    sim = lax.dot_general(anchors, tile, (((0,), (0,)), ((), ())),
                          preferred_element_type=jnp.float32)
    iota = lax.broadcasted_iota(jnp.int32, (M, TL), 0).astype(jnp.float32)
    acc = jnp.broadcast_to(b_ref[...], (out_c, TL)).astype(jnp.float32)
    for k in range(K):
        m = jnp.max(sim, axis=0, keepdims=True)
        cand = jnp.where(sim == m, iota, float(M))
        am = jnp.min(cand, axis=0, keepdims=True)
        sel = iota == am
        selb = sel.astype(jnp.bfloat16)
        for t_ref in (t1_ref, t2_ref, t3_ref):
            acc = acc + jnp.dot(t_ref[k], selb,
                                preferred_element_type=jnp.float32)
        sim = jnp.where(sel, -jnp.inf, sim)
    o_ref[0] = jnp.maximum(acc, 0.0) if relu else acc


def _layer(xf, anchors, W, b, relu, interpret=False):
    Bn, C, _ = xf.shape
    O = W.shape[0]
    wk = jnp.transpose(W, (2, 0, 1))
    grid = (Bn, L // TL)
    return pl.pallas_call(
        functools.partial(_layer_body, out_c=O, relu=relu),
        grid=grid,
        in_specs=[
            pl.BlockSpec((1, C, TL), lambda b_, t: (b_, 0, t)),
            pl.BlockSpec((1, C, M), lambda b_, t: (b_, 0, 0)),
            pl.BlockSpec((K, O, C), lambda b_, t: (0, 0, 0)),
            pl.BlockSpec((O, 1), lambda b_, t: (0, 0)),
        ],
        out_specs=pl.BlockSpec((1, O, TL), lambda b_, t: (b_, 0, t)),
        out_shape=jax.ShapeDtypeStruct((Bn, O, L), jnp.float32),
        scratch_shapes=[pltpu.VMEM((K, O, M), jnp.bfloat16),
                        pltpu.VMEM((K, O, M), jnp.bfloat16),
                        pltpu.VMEM((K, O, M), jnp.bfloat16)],
        interpret=interpret,
    )(xf, anchors, wk, b.reshape(O, 1))


def _run(x, W1, b1, W2, b2, W3, b3, interpret=False):
    B = x.shape[0]
    u = x.reshape(B, 3, H, SCALE, H, SCALE)
    u = u.transpose(0, 1, 3, 5, 2, 4).reshape(B, 12, L)
    sidx = _sample_idx()
    h = u
    for W, b, relu in ((W1, b1, True), (W2, b2, True), (W3, b3, False)):
        anchors = h[:, :, sidx]
        h = _layer(h, anchors, W, b, relu, interpret=interpret)
    out = h.reshape(B, 3, SCALE, SCALE, H, H)
    out = out.transpose(0, 1, 4, 2, 5, 3).reshape(B, 3, H * SCALE, H * SCALE)
    return out


def kernel(x, W1, b1, W2, b2, W3, b3):
    return _run(x, W1, b1, W2, b2, W3, b3)
